# Initial kernel scaffold; baseline (speedup 1.0000x reference)
import jax
import jax.numpy as jnp
from jax.experimental import pallas as pl


def kernel(x, edge_index, W1, att_src1, att_dst1, b1, W2, att_src2, att_dst2, b2):
    raise NotImplementedError("write your pallas kernel here")


# speedup vs baseline: 61.4360x; 61.4360x over previous
import functools

import jax
import jax.numpy as jnp
from jax import lax
from jax.experimental import pallas as pl
from jax.experimental.pallas import tpu as pltpu
from jax.experimental.pallas import tpu_sc as plsc

N = 10000
E = 320000
D = 128
HEADS = 8
HID = 16
NCLS = 16

NW = 32
EPW = E // NW
B = 80
NB = EPW // B
G = 5
NPAD = 10240
STRIPE = NPAD // 16


def _make_edge_pass(Dh, heads):
    nd = Dh // 16
    mesh = plsc.VectorSubcoreMesh(core_axis_name="c", subcore_axis_name="s")

    @functools.partial(
        pl.kernel,
        out_type=jax.ShapeDtypeStruct((2, NPAD, Dh), jnp.float32),
        mesh=mesh,
        scratch_types=[
            pltpu.VMEM((G, B), jnp.int32),
            pltpu.VMEM((G, B), jnp.int32),
            pltpu.VMEM((B, Dh), jnp.float32),
            pltpu.VMEM((B, 16), jnp.float32),
            pltpu.VMEM((B, Dh), jnp.float32),
            pltpu.VMEM_SHARED((NPAD, Dh), jnp.float32),
            pltpu.SemaphoreType.DMA,
            pltpu.SemaphoreType.DMA,
        ],
        compiler_params=pltpu.CompilerParams(use_tc_tiling_on_sc=False),
    )
    def edge_pass(hs_hbm, ad_hbm, src_hbm, dst_hbm, out_hbm,
                  sidx, didx, hsbuf, adbuf, msg, acc, sem1, sem2):
        cid = lax.axis_index("c")
        sid = lax.axis_index("s")
        wid = cid * 16 + sid

        def zrow(r, carry):
            for d_ in range(nd):
                msg[r, pl.ds(d_ * 16, 16)] = jnp.zeros((16,), jnp.float32)
            return carry
        lax.fori_loop(0, B, zrow, 0)

        def zcp(j, carry):
            pltpu.sync_copy(msg, acc.at[pl.ds(sid * STRIPE + j * B, B)])
            return carry
        lax.fori_loop(0, STRIPE // B, zcp, 0)
        plsc.subcore_barrier()

        def group(g, carry):
            cps = pltpu.async_copy(src_hbm.at[wid, pl.ds(g * G, G)], sidx, sem1)
            cpd = pltpu.async_copy(dst_hbm.at[wid, pl.ds(g * G, G)], didx, sem2)
            cps.wait()
            cpd.wait()

            def block(j, c1):
                cp1 = pltpu.async_copy(hs_hbm.at[sidx.at[j]], hsbuf, sem1)
                cp2 = pltpu.async_copy(ad_hbm.at[didx.at[j]], adbuf, sem2)
                cp1.wait()
                cp2.wait()

                def edge(e, c2):
                    ad = adbuf[e]
                    asv = hsbuf[e, pl.ds(Dh - 16, 16)]
                    ev = asv + ad
                    ev = jnp.where(ev >= 0.0, ev, 0.2 * ev)
                    ex = jnp.exp(ev)
                    for hd in range(heads):
                        exb = jnp.broadcast_to(ex[hd], (16,))
                        msg[e, pl.ds(hd * 16, 16)] = hsbuf[e, pl.ds(hd * 16, 16)] * exb
                    msg[e, pl.ds(Dh - 16, 16)] = ex
                    return c2
                lax.fori_loop(0, B, edge, 0)

                pltpu.sync_copy(msg, acc.at[didx.at[j]], add=True)
                return c1
            lax.fori_loop(0, G, block, 0)
            return carry
        lax.fori_loop(0, NB // G, group, 0)

        plsc.subcore_barrier()

        def wb(j, carry):
            r0 = sid * STRIPE + j * B
            pltpu.sync_copy(acc.at[pl.ds(r0, B)], msg)
            pltpu.sync_copy(msg, out_hbm.at[cid, pl.ds(r0, B)])
            return carry
        lax.fori_loop(0, STRIPE // B, wb, 0)

    return edge_pass


_edge_pass_1 = _make_edge_pass(144, HEADS)
_edge_pass_2 = _make_edge_pass(32, 1)


def _tc1(x, Wh, Wd):
    R = 1000

    def body(x_ref, wh_ref, wd_ref, o1_ref, o2_ref):
        xb = x_ref[...]
        o1_ref[...] = jnp.dot(xb, wh_ref[...], preferred_element_type=jnp.float32)
        o2_ref[...] = jnp.dot(xb, wd_ref[...], preferred_element_type=jnp.float32)

    return pl.pallas_call(
        body,
        grid=(N // R,),
        in_specs=[
            pl.BlockSpec((R, D), lambda i: (i, 0)),
            pl.BlockSpec((D, 144), lambda i: (0, 0)),
            pl.BlockSpec((D, 16), lambda i: (0, 0)),
        ],
        out_specs=[
            pl.BlockSpec((R, 144), lambda i: (i, 0)),
            pl.BlockSpec((R, 16), lambda i: (i, 0)),
        ],
        out_shape=[
            jax.ShapeDtypeStruct((N, 144), jnp.float32),
            jax.ShapeDtypeStruct((N, 16), jnp.float32),
        ],
    )(x, Wh, Wd)


def _tc2(p, b1r, Wh, Wd):
    R = 1000

    def body(p_ref, b_ref, wh_ref, wd_ref, o1_ref, o2_ref):
        acc = p_ref[0] + p_ref[1]
        num = acc[:, :128]
        den = acc[:, 128:136]
        col = lax.broadcasted_iota(jnp.int32, (8, 128), 1) // 16
        row = lax.broadcasted_iota(jnp.int32, (8, 128), 0)
        expand = (col == row).astype(jnp.float32)
        dexp = jnp.dot(den, expand, preferred_element_type=jnp.float32)
        x1 = num / (dexp + 1e-16) + b_ref[...]
        x1 = jnp.where(x1 > 0.0, x1, jnp.exp(x1) - 1.0)
        o1_ref[...] = jnp.dot(x1, wh_ref[...], preferred_element_type=jnp.float32)
        o2_ref[...] = jnp.dot(x1, wd_ref[...], preferred_element_type=jnp.float32)

    return pl.pallas_call(
        body,
        grid=(N // R,),
        in_specs=[
            pl.BlockSpec((2, R, 144), lambda i: (0, i, 0)),
            pl.BlockSpec((1, D), lambda i: (0, 0)),
            pl.BlockSpec((D, 32), lambda i: (0, 0)),
            pl.BlockSpec((D, 16), lambda i: (0, 0)),
        ],
        out_specs=[
            pl.BlockSpec((R, 32), lambda i: (i, 0)),
            pl.BlockSpec((R, 16), lambda i: (i, 0)),
        ],
        out_shape=[
            jax.ShapeDtypeStruct((N, 32), jnp.float32),
            jax.ShapeDtypeStruct((N, 16), jnp.float32),
        ],
    )(p, b1r, Wh, Wd)


def _tc3(p, b2r):
    R = 1000

    def body(p_ref, b_ref, o_ref):
        acc = p_ref[0] + p_ref[1]
        num = acc[:, :16]
        den = acc[:, 16:17]
        logits = num / (den + 1e-16) + b_ref[...]
        m = jnp.max(logits, axis=1, keepdims=True)
        s = logits - m
        lse = jnp.log(jnp.sum(jnp.exp(s), axis=1, keepdims=True))
        o_ref[...] = s - lse

    return pl.pallas_call(
        body,
        grid=(N // R,),
        in_specs=[
            pl.BlockSpec((2, R, 32), lambda i: (0, i, 0)),
            pl.BlockSpec((1, NCLS), lambda i: (0, 0)),
        ],
        out_specs=pl.BlockSpec((R, NCLS), lambda i: (i, 0)),
        out_shape=jax.ShapeDtypeStruct((N, NCLS), jnp.float32),
    )(p, b2r)


def kernel(x, edge_index, W1, att_src1, att_dst1, b1, W2, att_src2, att_dst2, b2):
    ei = edge_index.astype(jnp.int32)
    src3 = ei[0].reshape(NW, NB, B)
    dst3 = ei[1].reshape(NW, NB, B)

    W1r = W1.reshape(D, HEADS, HID)
    a_s = jnp.einsum("fhk,hk->fh", W1r, att_src1)
    a_d = jnp.einsum("fhk,hk->fh", W1r, att_dst1)
    z8 = jnp.zeros((D, 8), jnp.float32)
    W1hs = jnp.concatenate([W1, a_s, z8], axis=1)
    W1ad = jnp.concatenate([a_d, z8], axis=1)

    ws2 = W2 @ att_src2[0]
    wd2 = W2 @ att_dst2[0]
    z15 = jnp.zeros((D, 15), jnp.float32)
    W2hs = jnp.concatenate([W2, ws2[:, None], z15], axis=1)
    W2ad = jnp.concatenate([wd2[:, None], z15], axis=1)

    hs1, ad1 = _tc1(x, W1hs, W1ad)
    p1 = _edge_pass_1(hs1, ad1, src3, dst3)
    hs2, ad2 = _tc2(p1, b1.reshape(1, D), W2hs, W2ad)
    p2 = _edge_pass_2(hs2, ad2, src3, dst3)
    return _tc3(p2, b2.reshape(1, NCLS))
